# Initial kernel scaffold; baseline (speedup 1.0000x reference)
import jax
import jax.numpy as jnp
from jax.experimental import pallas as pl


def kernel(q, kv_cache, topk_indices, attn_sink):
    raise NotImplementedError("write your pallas kernel here")


# speedup vs baseline: 3.3458x; 3.3458x over previous
import functools

import jax
import jax.numpy as jnp
from jax import lax
from jax.experimental import pallas as pl
from jax.experimental.pallas import tpu as pltpu
from jax.experimental.pallas import tpu_sc as plsc

SCALE_Q = 0.041666666666666664
DV_LATENT = 512


@functools.lru_cache(maxsize=None)
def _make_sc_gather(S, D, R):
    info = plsc.get_sparse_core_info()
    nw = info.num_cores * info.num_subcores
    assert R % nw == 0
    rows_per_w = R // nw
    chunk = 128
    assert rows_per_w % chunk == 0
    n_chunks = rows_per_w // chunk
    mesh = plsc.VectorSubcoreMesh(core_axis_name="c", subcore_axis_name="s")

    @functools.partial(
        pl.kernel,
        mesh=mesh,
        out_type=jax.ShapeDtypeStruct((R, D), jnp.float32),
        scratch_types=[
            pltpu.VMEM((chunk,), jnp.int32),
            pltpu.VMEM((chunk, D), jnp.float32),
            pltpu.SemaphoreType.DMA,
        ],
        compiler_params=pltpu.CompilerParams(use_tc_tiling_on_sc=False),
    )
    def gather_k(cache_hbm, idx_hbm, out_hbm, idx_v, rows_v, sem):
        wid = lax.axis_index("s") * info.num_cores + lax.axis_index("c")
        base = wid * rows_per_w

        def body(j, carry):
            start = base + j * chunk
            pltpu.sync_copy(idx_hbm.at[pl.ds(start, chunk)], idx_v)
            pltpu.async_copy(cache_hbm.at[idx_v], rows_v, sem).wait()
            pltpu.sync_copy(rows_v, out_hbm.at[pl.ds(start, chunk)])
            return carry

        lax.fori_loop(0, n_chunks, body, 0)

    return gather_k


def _attn_body(q_ref, k_ref, sink_ref, o_ref):
    q = q_ref[0]
    k = k_ref[0]
    s = sink_ref[...]
    logits = lax.dot_general(
        q, k, (((1,), (1,)), ((), ())), preferred_element_type=jnp.float32
    ) * SCALE_Q
    m = jnp.maximum(jnp.max(logits, axis=1, keepdims=True), s)
    p = jnp.exp(logits - m)
    denom = jnp.sum(p, axis=1, keepdims=True) + jnp.exp(s - m)
    attn = p / denom
    v = k[:, :DV_LATENT]
    o_ref[0] = lax.dot_general(
        attn, v, (((1,), (0,)), ((), ())), preferred_element_type=jnp.float32
    )


def _tc_attn(q, k3, sink, interpret=False):
    T, H, D = q.shape
    K = k3.shape[1]
    return pl.pallas_call(
        _attn_body,
        grid=(T,),
        in_specs=[
            pl.BlockSpec((1, H, D), lambda t: (t, 0, 0)),
            pl.BlockSpec((1, K, D), lambda t: (t, 0, 0)),
            pl.BlockSpec((H, 1), lambda t: (0, 0)),
        ],
        out_specs=pl.BlockSpec((1, H, DV_LATENT), lambda t: (t, 0, 0)),
        out_shape=jax.ShapeDtypeStruct((T, H, DV_LATENT), jnp.float32),
        interpret=interpret,
    )(q, k3, sink)


def kernel(q, kv_cache, topk_indices, attn_sink):
    T, H, D = q.shape
    K = topk_indices.shape[1]
    S = kv_cache.shape[0]
    idx_flat = topk_indices.reshape(-1)
    gathered = _make_sc_gather(S, D, T * K)(kv_cache, idx_flat)
    return _tc_attn(q, gathered.reshape(T, K, D), attn_sink.reshape(H, 1))


# speedup vs baseline: 3.3465x; 1.0002x over previous
import functools

import jax
import jax.numpy as jnp
from jax import lax
from jax.experimental import pallas as pl
from jax.experimental.pallas import tpu as pltpu
from jax.experimental.pallas import tpu_sc as plsc

SCALE_Q = 0.041666666666666664
DV_LATENT = 512


@functools.lru_cache(maxsize=None)
def _make_sc_gather(S, D, R):
    info = plsc.get_sparse_core_info()
    nw = info.num_cores * info.num_subcores
    assert R % nw == 0
    rows_per_w = R // nw
    chunk = 128
    assert rows_per_w % chunk == 0
    n_chunks = rows_per_w // chunk
    mesh = plsc.VectorSubcoreMesh(core_axis_name="c", subcore_axis_name="s")

    @functools.partial(
        pl.kernel,
        mesh=mesh,
        out_type=jax.ShapeDtypeStruct((R, D), jnp.float32),
        scratch_types=[
            pltpu.VMEM((chunk,), jnp.int32),
            pltpu.VMEM((chunk, D), jnp.float32),
            pltpu.SemaphoreType.DMA,
        ],
        compiler_params=pltpu.CompilerParams(use_tc_tiling_on_sc=False),
    )
    def gather_k(cache_hbm, idx_hbm, out_hbm, idx_v, rows_v, sem):
        wid = lax.axis_index("s") * info.num_cores + lax.axis_index("c")
        base = wid * rows_per_w

        def body(j, carry):
            start = base + j * chunk
            pltpu.sync_copy(idx_hbm.at[pl.ds(start, chunk)], idx_v)
            pltpu.async_copy(cache_hbm.at[idx_v], rows_v, sem).wait()
            pltpu.sync_copy(rows_v, out_hbm.at[pl.ds(start, chunk)])
            return carry

        lax.fori_loop(0, n_chunks, body, 0)

    return gather_k


def _attn_body(q_ref, k_ref, sink_ref, o_ref):
    q = q_ref[0].astype(jnp.bfloat16)
    k = k_ref[0].astype(jnp.bfloat16)
    s = sink_ref[...]
    logits = lax.dot_general(
        q, k, (((1,), (1,)), ((), ())), preferred_element_type=jnp.float32
    ) * SCALE_Q
    m = jnp.maximum(jnp.max(logits, axis=1, keepdims=True), s)
    p = jnp.exp(logits - m)
    denom = jnp.sum(p, axis=1, keepdims=True) + jnp.exp(s - m)
    attn = (p / denom).astype(jnp.bfloat16)
    v = k[:, :DV_LATENT]
    o_ref[0] = lax.dot_general(
        attn, v, (((1,), (0,)), ((), ())), preferred_element_type=jnp.float32
    )


def _tc_attn(q, k3, sink, interpret=False):
    T, H, D = q.shape
    K = k3.shape[1]
    return pl.pallas_call(
        _attn_body,
        grid=(T,),
        in_specs=[
            pl.BlockSpec((1, H, D), lambda t: (t, 0, 0)),
            pl.BlockSpec((1, K, D), lambda t: (t, 0, 0)),
            pl.BlockSpec((H, 1), lambda t: (0, 0)),
        ],
        out_specs=pl.BlockSpec((1, H, DV_LATENT), lambda t: (t, 0, 0)),
        out_shape=jax.ShapeDtypeStruct((T, H, DV_LATENT), jnp.float32),
        interpret=interpret,
    )(q, k3, sink)


def kernel(q, kv_cache, topk_indices, attn_sink):
    T, H, D = q.shape
    K = topk_indices.shape[1]
    S = kv_cache.shape[0]
    idx_flat = topk_indices.reshape(-1)
    gathered = _make_sc_gather(S, D, T * K)(kv_cache, idx_flat)
    return _tc_attn(q, gathered.reshape(T, K, D), attn_sink.reshape(H, 1))


# speedup vs baseline: 5.4803x; 1.6376x over previous
import functools

import jax
import jax.numpy as jnp
from jax import lax
from jax.experimental import pallas as pl
from jax.experimental.pallas import tpu as pltpu
from jax.experimental.pallas import tpu_sc as plsc

SCALE_Q = 0.041666666666666664
DV_LATENT = 512
D_PAD = 640


@functools.lru_cache(maxsize=None)
def _make_sc_gather(S, T, K):
    info = plsc.get_sparse_core_info()
    nw = info.num_cores * info.num_subcores
    R = T * K
    assert R % nw == 0
    rows_per_w = R // nw
    chunk = 128
    assert rows_per_w % chunk == 0 and K % chunk == 0
    n_chunks = rows_per_w // chunk
    chunks_per_tok = K // chunk
    mesh = plsc.VectorSubcoreMesh(core_axis_name="c", subcore_axis_name="s")

    @functools.partial(
        pl.kernel,
        mesh=mesh,
        out_type=jax.ShapeDtypeStruct((R, D_PAD), jnp.float32),
        scratch_types=[
            pltpu.VMEM((1, chunk), jnp.int32),
            pltpu.VMEM((chunk, D_PAD), jnp.float32),
            pltpu.SemaphoreType.DMA,
        ],
    )
    def gather_k(cache_hbm, idx_hbm, out_hbm, idx_v, rows_v, sem):
        wid = lax.axis_index("s") * info.num_cores + lax.axis_index("c")
        base = wid * rows_per_w

        def body(j, carry):
            row0 = base + j * chunk
            tok = row0 // K
            col = row0 % K
            pltpu.sync_copy(idx_hbm.at[pl.ds(tok, 1), pl.ds(col, chunk)], idx_v)
            pltpu.async_copy(cache_hbm.at[idx_v.at[0]], rows_v, sem).wait()
            pltpu.sync_copy(rows_v, out_hbm.at[pl.ds(row0, chunk)])
            return carry

        lax.fori_loop(0, n_chunks, body, 0)

    return gather_k


def _attn_body(q_ref, k_ref, sink_ref, o_ref):
    q = q_ref[0].astype(jnp.bfloat16)
    kb = k_ref[...].astype(jnp.bfloat16)
    s = sink_ref[...]
    logits = lax.dot_general(
        q, kb, (((1,), (1,)), ((), ())), preferred_element_type=jnp.float32
    ) * SCALE_Q
    m = jnp.maximum(jnp.max(logits, axis=1, keepdims=True), s)
    p = jnp.exp(logits - m)
    denom = jnp.sum(p, axis=1, keepdims=True) + jnp.exp(s - m)
    attn = (p / denom).astype(jnp.bfloat16)
    v = kb[:, :DV_LATENT]
    o_ref[0] = lax.dot_general(
        attn, v, (((1,), (0,)), ((), ())), preferred_element_type=jnp.float32
    )


def _tc_attn(q, gathered, sink, interpret=False):
    T, H, D = q.shape
    K = gathered.shape[0] // T
    return pl.pallas_call(
        _attn_body,
        grid=(T,),
        in_specs=[
            pl.BlockSpec((1, H, D), lambda t: (t, 0, 0)),
            pl.BlockSpec((K, D_PAD), lambda t: (t, 0)),
            pl.BlockSpec((H, 1), lambda t: (0, 0)),
        ],
        out_specs=pl.BlockSpec((1, H, DV_LATENT), lambda t: (t, 0, 0)),
        out_shape=jax.ShapeDtypeStruct((T, H, DV_LATENT), jnp.float32),
        interpret=interpret,
    )(q, gathered, sink)


def kernel(q, kv_cache, topk_indices, attn_sink):
    T, H, D = q.shape
    K = topk_indices.shape[1]
    S = kv_cache.shape[0]
    cache_p = jnp.pad(kv_cache, ((0, 0), (0, D_PAD - D)))
    q_p = jnp.pad(q, ((0, 0), (0, 0), (0, D_PAD - D)))
    gathered = _make_sc_gather(S, T, K)(cache_p, topk_indices)
    return _tc_attn(q_p, gathered, attn_sink.reshape(H, 1))


# speedup vs baseline: 5.7575x; 1.0506x over previous
import functools

import jax
import jax.numpy as jnp
from jax import lax
from jax.experimental import pallas as pl
from jax.experimental.pallas import tpu as pltpu
from jax.experimental.pallas import tpu_sc as plsc

SCALE_Q = 0.041666666666666664
DV_LATENT = 512
D_PAD = 640
N_CHUNKS_T = 4


@functools.lru_cache(maxsize=None)
def _make_sc_gather(S, T, K):
    info = plsc.get_sparse_core_info()
    nw = info.num_cores * info.num_subcores
    R = T * K
    assert R % nw == 0
    rows_per_w = R // nw
    chunk = 64
    assert rows_per_w % (2 * chunk) == 0 and K % chunk == 0
    n_pairs = rows_per_w // (2 * chunk)
    mesh = plsc.VectorSubcoreMesh(core_axis_name="c", subcore_axis_name="s")

    @functools.partial(
        pl.kernel,
        mesh=mesh,
        out_type=jax.ShapeDtypeStruct((R, D_PAD), jnp.float32),
        scratch_types=[
            pltpu.VMEM((1, 2 * chunk), jnp.int32),
            pltpu.VMEM((chunk, D_PAD), jnp.float32),
            pltpu.VMEM((chunk, D_PAD), jnp.float32),
            pltpu.SemaphoreType.DMA,
            pltpu.SemaphoreType.DMA,
            pltpu.SemaphoreType.DMA,
        ],
    )
    def gather_k(cache_hbm, idx_hbm, out_hbm, idx_v, rows_v0,
                 rows_v1, sem_g, sem_w0, sem_w1):
        wid = lax.axis_index("s") * info.num_cores + lax.axis_index("c")
        base = wid * rows_per_w
        bufs = ((rows_v0, sem_w0), (rows_v1, sem_w1))

        def body(i, carry):
            pair0 = base + i * 2 * chunk
            tok = pair0 // K
            col = pair0 % K
            pltpu.sync_copy(
                idx_hbm.at[pl.ds(tok, 1), pl.ds(col, 2 * chunk)], idx_v
            )
            for b in range(2):
                rows_v, sem_w = bufs[b]
                row0 = pair0 + b * chunk

                @pl.when(i > 0)
                def _wait_prev():
                    pltpu.make_async_copy(
                        rows_v, out_hbm.at[pl.ds(base, chunk)], sem_w
                    ).wait()

                pltpu.async_copy(
                    cache_hbm.at[idx_v.at[0, pl.ds(b * chunk, chunk)]],
                    rows_v, sem_g,
                ).wait()
                pltpu.async_copy(rows_v, out_hbm.at[pl.ds(row0, chunk)], sem_w)
            return carry

        lax.fori_loop(0, n_pairs, body, 0)
        for b in range(2):
            rows_v, sem_w = bufs[b]
            pltpu.make_async_copy(
                rows_v, out_hbm.at[pl.ds(base, chunk)], sem_w
            ).wait()

    return gather_k


def _attn_body(q_ref, k_ref, sink_ref, o_ref):
    q = q_ref[0].astype(jnp.bfloat16)
    kb = k_ref[...].astype(jnp.bfloat16)
    s = sink_ref[...]
    logits = lax.dot_general(
        q, kb, (((1,), (1,)), ((), ())), preferred_element_type=jnp.float32
    ) * SCALE_Q
    m = jnp.maximum(jnp.max(logits, axis=1, keepdims=True), s)
    p = jnp.exp(logits - m)
    denom = jnp.sum(p, axis=1, keepdims=True) + jnp.exp(s - m)
    attn = (p / denom).astype(jnp.bfloat16)
    v = kb[:, :DV_LATENT]
    o_ref[0] = lax.dot_general(
        attn, v, (((1,), (0,)), ((), ())), preferred_element_type=jnp.float32
    )


def _tc_attn(q, gathered, sink, interpret=False):
    T, H, D = q.shape
    K = gathered.shape[0] // T
    return pl.pallas_call(
        _attn_body,
        grid=(T,),
        in_specs=[
            pl.BlockSpec((1, H, D), lambda t: (t, 0, 0)),
            pl.BlockSpec((K, D_PAD), lambda t: (t, 0)),
            pl.BlockSpec((H, 1), lambda t: (0, 0)),
        ],
        out_specs=pl.BlockSpec((1, H, DV_LATENT), lambda t: (t, 0, 0)),
        out_shape=jax.ShapeDtypeStruct((T, H, DV_LATENT), jnp.float32),
        interpret=interpret,
    )(q, gathered, sink)


def kernel(q, kv_cache, topk_indices, attn_sink):
    T, H, D = q.shape
    K = topk_indices.shape[1]
    S = kv_cache.shape[0]
    cache_p = jnp.pad(kv_cache, ((0, 0), (0, D_PAD - D)))
    q_p = jnp.pad(q, ((0, 0), (0, 0), (0, D_PAD - D)))
    sink = attn_sink.reshape(H, 1)
    tc = T // N_CHUNKS_T
    gather = _make_sc_gather(S, tc, K)
    outs = []
    for c in range(N_CHUNKS_T):
        g = gather(cache_p, topk_indices[c * tc:(c + 1) * tc])
        outs.append(_tc_attn(q_p[c * tc:(c + 1) * tc], g, sink))
    return jnp.concatenate(outs, axis=0)
